# baseline (speedup 1.0000x reference)
import functools

import jax
import jax.numpy as jnp
from jax import lax
from jax.experimental import pallas as pl
from jax.experimental.pallas import tpu as pltpu
from jax.experimental.pallas import tpu_sc as plsc

BATCH = 4096
SEQ = 200
HIDDEN = 64
PADH = 128
B = BATCH * SEQ
NC = 2
NS = 16
NW = NC * NS
B_PER_W = B // NW
IDX_ROW = 128
ROWS_PER_BLOCK = 512
GPB = ROWS_PER_BLOCK // IDX_ROW
NBLK = B_PER_W // ROWS_PER_BLOCK
IDS_ROWS_PER_W = B_PER_W // IDX_ROW
NBUF = 2


def kernel(input_ids, embed):
    ids2d = input_ids.reshape(B // IDX_ROW, IDX_ROW)
    mesh = plsc.VectorSubcoreMesh(core_axis_name="c", subcore_axis_name="s")

    @functools.partial(
        pl.kernel,
        mesh=mesh,
        out_type=jax.ShapeDtypeStruct((B, PADH), jnp.float32),
        compiler_params=pltpu.CompilerParams(
            use_tc_tiling_on_sc=False, needs_layout_passes=False
        ),
        scratch_types=[
            pltpu.VMEM((IDS_ROWS_PER_W, IDX_ROW), jnp.int32),
            pltpu.VMEM((NBUF, ROWS_PER_BLOCK, HIDDEN), jnp.float32),
            pltpu.SemaphoreType.DMA((NBUF,)),
            pltpu.SemaphoreType.DMA((NBUF,)),
        ],
    )
    def emb(ids_hbm, table_hbm, out_hbm, idx_v, rows_v, gsem, ssem):
        wid = lax.axis_index("s") * NC + lax.axis_index("c")
        id_row0 = wid * IDS_ROWS_PER_W
        out0 = wid * B_PER_W

        pltpu.sync_copy(ids_hbm.at[pl.ds(id_row0, IDS_ROWS_PER_W)], idx_v)

        def fire_gathers(g, b):
            for j in range(GPB):
                pltpu.async_copy(
                    table_hbm.at[idx_v.at[g * GPB + j]],
                    rows_v.at[b, pl.ds(j * IDX_ROW, IDX_ROW)],
                    gsem.at[b],
                )

        def drain_gathers(b):
            pltpu.make_async_copy(
                table_hbm.at[pl.ds(0, ROWS_PER_BLOCK)], rows_v.at[b],
                gsem.at[b],
            ).wait()

        def fire_store(g, b):
            pltpu.async_copy(
                rows_v.at[b],
                out_hbm.at[
                    pl.ds(out0 + g * ROWS_PER_BLOCK, ROWS_PER_BLOCK),
                    pl.ds(0, HIDDEN),
                ],
                ssem.at[b],
            )

        def drain_store(b):
            pltpu.make_async_copy(
                rows_v.at[b],
                out_hbm.at[pl.ds(out0, ROWS_PER_BLOCK), pl.ds(0, HIDDEN)],
                ssem.at[b],
            ).wait()

        for g in range(NBUF - 1):
            fire_gathers(g, g % NBUF)

        def step(o, carry):
            for b in range(NBUF):
                s = o * NBUF + b
                drain_gathers(b)
                fire_store(s, b)
                pb = (b - 1) % NBUF
                fb = s + NBUF - 1

                @pl.when(fb < NBLK)
                def _fire():
                    @pl.when(fb >= NBUF)
                    def _wait_prev_store():
                        drain_store(pb)

                    fire_gathers(fb, pb)

            return carry

        lax.fori_loop(0, NBLK // NBUF, step, 0)

        for b in range(NBUF):
            drain_store(b)

    out = emb(ids2d, embed)
    return out[:, :HIDDEN].reshape(BATCH, SEQ, HIDDEN)
